# Initial kernel scaffold; baseline (speedup 1.0000x reference)
import jax
import jax.numpy as jnp
from jax.experimental import pallas as pl


def kernel(edge_index, node, eft, target_idx):
    raise NotImplementedError("write your pallas kernel here")


# speedup vs baseline: 8.9249x; 8.9249x over previous
import functools

import jax
import jax.numpy as jnp
from jax import lax
from jax.experimental import pallas as pl
from jax.experimental.pallas import tpu as pltpu
from jax.experimental.pallas import tpu_sc as plsc

N_NODES = 10000
E_EDGES = 320000
H = 8
D = 16
HD = H * D
B_TGT = 1024

NC = 2
NS = 16
L = 16
NW = NC * NS
EPW = E_EDGES // NW
EB = 80
NBLK = EPW // EB
NG = EB // L
ROWS_PT = 624
ROWS_LAST = N_NODES - (NS - 1) * ROWS_PT
TPW = B_TGT // NW


def _mesh():
    return plsc.VectorSubcoreMesh(
        core_axis_name="c", subcore_axis_name="s",
        num_cores=NC, num_subcores=NS)


def _acc_chunks(total):
    out = []
    off = 0
    while off < total:
        n = min(EB, total - off)
        out.append((off, n))
        off += n
    return out


@functools.partial(
    pl.kernel,
    out_type=(
        jax.ShapeDtypeStruct((E_EDGES * H,), jnp.float32),
        jax.ShapeDtypeStruct((NC, N_NODES, HD), jnp.float32),
    ),
    mesh=_mesh(),
    scratch_types=[
        pltpu.VMEM_SHARED((N_NODES, HD), jnp.float32),
        pltpu.VMEM((EB * HD,), jnp.float32),
        pltpu.VMEM((EB, HD), jnp.float32),
        pltpu.VMEM((EB * H,), jnp.float32),
        pltpu.VMEM((EB,), jnp.int32),
        pltpu.VMEM((1, EB), jnp.int32),
        pltpu.SemaphoreType.DMA,
    ],
    compiler_params=pltpu.CompilerParams(needs_layout_passes=False),
)
def _phase1(dst_hbm, node_hbm, eft_hbm, a_hbm, part_hbm,
            acc, eftv, nrow, av, dstv, idx2, sem):
    c = lax.axis_index("c")
    s = lax.axis_index("s")
    zero16 = jnp.zeros((L,), jnp.float32)
    iota = lax.iota(jnp.int32, L)

    def zero_body(i, carry):
        for j in range(H):
            nrow[i, pl.ds(j * L, L)] = zero16
        return carry
    lax.fori_loop(0, EB, zero_body, 0)
    r0 = s * ROWS_PT

    @pl.when(s < NS - 1)
    def _():
        for off, n in _acc_chunks(ROWS_PT):
            pltpu.sync_copy(nrow.at[pl.ds(0, n)], acc.at[pl.ds(r0 + off, n)])

    @pl.when(s == NS - 1)
    def _():
        for off, n in _acc_chunks(ROWS_LAST):
            pltpu.sync_copy(nrow.at[pl.ds(0, n)], acc.at[pl.ds(r0 + off, n)])

    plsc.subcore_barrier()

    def block_body(blk, carry):
        base = c * (NS * EPW) + s * EPW + blk * EB
        pltpu.sync_copy(dst_hbm.at[pl.ds(base, EB)], dstv)
        pltpu.sync_copy(eft_hbm.at[pl.ds(base * HD, EB * HD)], eftv)
        gat = pltpu.async_copy(node_hbm.at[dstv], nrow, sem)
        for k in range(NG):
            idx2[0, pl.ds(k * L, L)] = dstv[pl.ds(k * L, L)]
        gat.wait()

        def group_body(g, gc):
            rows = g * L + iota
            ebase = rows * HD
            sims = []
            for h in range(H):
                acc_h = None
                for d in range(D):
                    col = h * D + d
                    et = plsc.load_gather(eftv, [ebase + col])
                    nt = plsc.load_gather(
                        nrow, [rows, jnp.full((L,), col, jnp.int32)])
                    p = et * nt
                    acc_h = p if acc_h is None else acc_h + p
                sims.append(acc_h)
            m = sims[0]
            for h in range(1, H):
                m = jnp.maximum(m, sims[h])
            zs = [jnp.exp(sims[h] - m) for h in range(H)]
            ssum = zs[0]
            for h in range(1, H):
                ssum = ssum + zs[h]
            rs = 1.0 / ssum
            a_hs = [zs[h] * rs for h in range(H)]
            abase = rows * H
            for h in range(H):
                plsc.store_scatter(av, [abase + h], a_hs[h])
            for h in range(H):
                for d in range(D):
                    col = h * D + d
                    et = plsc.load_gather(eftv, [ebase + col])
                    plsc.store_scatter(
                        nrow, [rows, jnp.full((L,), col, jnp.int32)],
                        et * a_hs[h])
            return gc
        lax.fori_loop(0, NG, group_body, 0)

        pltpu.sync_copy(av, a_hbm.at[pl.ds(base * H, EB * H)])
        pltpu.sync_copy(nrow, acc.at[idx2.at[0]], add=True)
        return carry
    lax.fori_loop(0, NBLK, block_body, 0)

    plsc.subcore_barrier()

    @pl.when(s < NS - 1)
    def _():
        for off, n in _acc_chunks(ROWS_PT):
            pltpu.sync_copy(acc.at[pl.ds(r0 + off, n)], nrow.at[pl.ds(0, n)])
            pltpu.sync_copy(nrow.at[pl.ds(0, n)],
                            part_hbm.at[c].at[pl.ds(r0 + off, n)])

    @pl.when(s == NS - 1)
    def _():
        for off, n in _acc_chunks(ROWS_LAST):
            pltpu.sync_copy(acc.at[pl.ds(r0 + off, n)], nrow.at[pl.ds(0, n)])
            pltpu.sync_copy(nrow.at[pl.ds(0, n)],
                            part_hbm.at[c].at[pl.ds(r0 + off, n)])


@functools.partial(
    pl.kernel,
    out_type=jax.ShapeDtypeStruct((B_TGT, HD), jnp.float32),
    mesh=_mesh(),
    scratch_types=[
        pltpu.VMEM((TPW,), jnp.int32),
        pltpu.VMEM((TPW, HD), jnp.float32),
        pltpu.VMEM((TPW, HD), jnp.float32),
        pltpu.VMEM((TPW, HD), jnp.float32),
        pltpu.SemaphoreType.DMA,
    ],
    compiler_params=pltpu.CompilerParams(needs_layout_passes=False),
)
def _phase2(p0_hbm, p1_hbm, tgt_hbm, out_hbm, idxv, r0v, r1v, ov, sem):
    c = lax.axis_index("c")
    s = lax.axis_index("s")
    base = (s * NC + c) * TPW
    pltpu.sync_copy(tgt_hbm.at[pl.ds(base, TPW)], idxv)
    pltpu.async_copy(p0_hbm.at[idxv], r0v, sem).wait()
    pltpu.async_copy(p1_hbm.at[idxv], r1v, sem).wait()

    def row_body(i, carry):
        vs = []
        for j in range(H):
            vs.append(r0v[i, pl.ds(j * D, D)] + r1v[i, pl.ds(j * D, D)]
                      + 1e-15)
        ssq = vs[0] * vs[0]
        for j in range(1, H):
            ssq = ssq + vs[j] * vs[j]
        ib = plsc.bitcast(ssq, jnp.int32)
        y = plsc.bitcast(jnp.int32(0x5F3759DF) - (ib >> 1), jnp.float32)
        for _ in range(3):
            y = y * (1.5 - 0.5 * ssq * y * y)
        y = jnp.minimum(y, 1e12)
        for j in range(H):
            ov[i, pl.ds(j * D, D)] = vs[j] * y
        return carry
    lax.fori_loop(0, TPW, row_body, 0)
    pltpu.sync_copy(ov, out_hbm.at[pl.ds(base, TPW)])


def kernel(edge_index, node, eft, target_idx):
    dst = edge_index[1]
    node2 = node.reshape(N_NODES, HD)
    eftf = eft.reshape(E_EDGES * HD)
    a_flat, parts = _phase1(dst, node2, eftf)
    out2 = _phase2(parts[0], parts[1], target_idx)
    return (out2.reshape(B_TGT, H, D), a_flat.reshape(E_EDGES, H, 1))


# speedup vs baseline: 13.9378x; 1.5617x over previous
import functools

import jax
import jax.numpy as jnp
from jax import lax
from jax.experimental import pallas as pl
from jax.experimental.pallas import tpu as pltpu
from jax.experimental.pallas import tpu_sc as plsc

N_NODES = 10000
E_EDGES = 320000
H = 8
D = 16
HD = H * D
B_TGT = 1024

NC = 2
NS = 16
L = 16
NW = NC * NS
EPW = E_EDGES // NW
EB = 400
NBLK = EPW // EB
NG = EB // L
CW = 64
NCW = 8
TRASH = B_TGT
ACC_ROWS = 1152
TPW = B_TGT // NW


def _mesh():
    return plsc.VectorSubcoreMesh(
        core_axis_name="c", subcore_axis_name="s",
        num_cores=NC, num_subcores=NS)


def _build_map(tgt_hbm, tgtb, nmap):
    pltpu.sync_copy(tgt_hbm, tgtb.at[pl.ds(0, B_TGT)])
    neg = jnp.full((L,), -1, jnp.int32)
    lane0 = lax.iota(jnp.int32, L) == 0

    def zbody(i, c):
        nmap[pl.ds(i * L, L)] = neg
        return c
    lax.fori_loop(0, N_NODES // L + 1, zbody, 0)

    def mbody(b, c):
        tv = tgtb[pl.ds(b, L)]
        plsc.store_scatter(nmap, [tv], jnp.full((L,), b, jnp.int32),
                           mask=lane0)
        return c
    lax.fori_loop(0, B_TGT, mbody, 0)


@functools.partial(
    pl.kernel,
    out_type=(
        jax.ShapeDtypeStruct((E_EDGES * H,), jnp.float32),
        jax.ShapeDtypeStruct((NC, B_TGT, HD), jnp.float32),
    ),
    mesh=_mesh(),
    scratch_types=[
        pltpu.VMEM_SHARED((ACC_ROWS, HD), jnp.float32),
        pltpu.VMEM((N_NODES + L,), jnp.int32),
        pltpu.VMEM((B_TGT + L,), jnp.int32),
        pltpu.VMEM((EB * HD,), jnp.float32),
        pltpu.VMEM((EB, HD), jnp.float32),
        pltpu.VMEM((EB * H,), jnp.float32),
        pltpu.VMEM((EB,), jnp.int32),
        pltpu.VMEM((EB,), jnp.int32),
        pltpu.VMEM((NCW, CW), jnp.int32),
        pltpu.SemaphoreType.DMA,
    ],
    compiler_params=pltpu.CompilerParams(needs_layout_passes=False),
)
def _phase1(dst_hbm, node_hbm, eft_hbm, tgt_hbm, a_hbm, part_hbm,
            acc, nmap, tgtb, eftv, nrow, av, dstv, hid, wl, sem):
    c = lax.axis_index("c")
    s = lax.axis_index("s")
    zero16 = jnp.zeros((L,), jnp.float32)
    iota = lax.iota(jnp.int32, L)

    _build_map(tgt_hbm, tgtb, nmap)

    zero16i = jnp.zeros((L,), jnp.int32)

    def hz(i, carry):
        hid[pl.ds(i * L, L)] = zero16i
        return carry
    lax.fori_loop(0, EB // L, hz, 0)

    def zrow(i, carry):
        for j in range(H):
            nrow[i, pl.ds(j * L, L)] = zero16
        return carry
    lax.fori_loop(0, ACC_ROWS // NS, zrow, 0)
    pltpu.sync_copy(nrow.at[pl.ds(0, ACC_ROWS // NS)],
                    acc.at[pl.ds(s * (ACC_ROWS // NS), ACC_ROWS // NS)])
    plsc.subcore_barrier()

    def block_body(blk, carry):
        base = c * (NS * EPW) + s * EPW + blk * EB
        pltpu.sync_copy(dst_hbm.at[pl.ds(base, EB)], dstv)
        gat = pltpu.async_copy(node_hbm.at[dstv], nrow, sem)
        pltpu.sync_copy(eft_hbm.at[pl.ds(base * HD, EB * HD)], eftv)
        gat.wait()

        def group_body(g, cnt):
            rows = g * L + iota
            ebase = rows * HD
            sims = []
            for h in range(H):
                acc_h = None
                for d in range(D):
                    col = h * D + d
                    et = plsc.load_gather(eftv, [ebase + col])
                    nt = plsc.load_gather(
                        nrow, [rows, jnp.full((L,), col, jnp.int32)])
                    p = et * nt
                    acc_h = p if acc_h is None else acc_h + p
                sims.append(acc_h)
            m = sims[0]
            for h in range(1, H):
                m = jnp.maximum(m, sims[h])
            zs = [jnp.exp(sims[h] - m) for h in range(H)]
            ssum = zs[0]
            for h in range(1, H):
                ssum = ssum + zs[h]
            rs = 1.0 / ssum
            a_hs = [zs[h] * rs for h in range(H)]
            abase = rows * H
            for h in range(H):
                plsc.store_scatter(av, [abase + h], a_hs[h])
            dvec = dstv[pl.ds(g * L, L)]
            slots = plsc.load_gather(nmap, [dvec])
            hit = slots >= 0
            hi = hit.astype(jnp.int32)
            pos = cnt + plsc.cumsum(hi) - hi
            plsc.store_scatter(hid, [pos], rows, mask=hit)
            plsc.store_scatter(wl, [pos >> 6, pos & (CW - 1)], slots,
                               mask=hit)
            return cnt + jnp.sum(hi)
        cnt = lax.fori_loop(0, NG, group_body, jnp.int32(0))

        trash = jnp.full((L,), TRASH, jnp.int32)
        for t in range(CW // L):
            p = cnt + t * L + iota
            plsc.store_scatter(wl, [p >> 6, p & (CW - 1)], trash)

        def hit_body(k, carry):
            drow = k * L + iota
            eids = plsc.load_gather(hid, [drow])
            ebase = eids * HD
            a_hs = [plsc.load_gather(av, [eids * H + h]) for h in range(H)]
            for h in range(H):
                for d in range(D):
                    col = h * D + d
                    et = plsc.load_gather(eftv, [ebase + col])
                    plsc.store_scatter(
                        nrow, [drow, jnp.full((L,), col, jnp.int32)],
                        et * a_hs[h])
            return carry
        lax.fori_loop(0, (cnt + L - 1) >> 4, hit_body, 0)

        pltpu.sync_copy(av, a_hbm.at[pl.ds(base * H, EB * H)])
        nchunk = (cnt + CW - 1) >> 6

        def sc_body(j, carry):
            pltpu.sync_copy(nrow.at[pl.ds(j * CW, CW)],
                            acc.at[wl.at[j]], add=True)
            return carry
        lax.fori_loop(0, nchunk, sc_body, 0)
        return 0
    lax.fori_loop(0, NBLK, block_body, 0)

    plsc.subcore_barrier()
    r0 = s * (B_TGT // NS)
    pltpu.sync_copy(acc.at[pl.ds(r0, B_TGT // NS)],
                    nrow.at[pl.ds(0, B_TGT // NS)])
    pltpu.sync_copy(nrow.at[pl.ds(0, B_TGT // NS)],
                    part_hbm.at[c].at[pl.ds(r0, B_TGT // NS)])


@functools.partial(
    pl.kernel,
    out_type=jax.ShapeDtypeStruct((B_TGT, HD), jnp.float32),
    mesh=_mesh(),
    scratch_types=[
        pltpu.VMEM((N_NODES + L,), jnp.int32),
        pltpu.VMEM((B_TGT + L,), jnp.int32),
        pltpu.VMEM((TPW,), jnp.int32),
        pltpu.VMEM((TPW, HD), jnp.float32),
        pltpu.VMEM((TPW, HD), jnp.float32),
        pltpu.VMEM((TPW, HD), jnp.float32),
        pltpu.SemaphoreType.DMA,
    ],
    compiler_params=pltpu.CompilerParams(needs_layout_passes=False),
)
def _phase2(p0_hbm, p1_hbm, tgt_hbm, out_hbm,
            nmap, tgtb, slotv, r0v, r1v, ov, sem):
    c = lax.axis_index("c")
    s = lax.axis_index("s")
    iota = lax.iota(jnp.int32, L)
    base = (s * NC + c) * TPW

    _build_map(tgt_hbm, tgtb, nmap)
    for k in range(TPW // L):
        tv = tgtb[pl.ds(base + k * L, L)]
        slotv[pl.ds(k * L, L)] = plsc.load_gather(nmap, [tv])
    pltpu.async_copy(p0_hbm.at[slotv], r0v, sem).wait()
    pltpu.async_copy(p1_hbm.at[slotv], r1v, sem).wait()

    def row_body(i, carry):
        vs = []
        for j in range(H):
            vs.append(r0v[i, pl.ds(j * D, D)] + r1v[i, pl.ds(j * D, D)]
                      + 1e-15)
        ssq = vs[0] * vs[0]
        for j in range(1, H):
            ssq = ssq + vs[j] * vs[j]
        ib = plsc.bitcast(ssq, jnp.int32)
        y = plsc.bitcast(jnp.int32(0x5F3759DF) - (ib >> 1), jnp.float32)
        for _ in range(3):
            y = y * (1.5 - 0.5 * ssq * y * y)
        y = jnp.minimum(y, 1e12)
        for j in range(H):
            ov[i, pl.ds(j * D, D)] = vs[j] * y
        return carry
    lax.fori_loop(0, TPW, row_body, 0)
    pltpu.sync_copy(ov, out_hbm.at[pl.ds(base, TPW)])


def kernel(edge_index, node, eft, target_idx):
    dst = edge_index[1]
    node2 = node.reshape(N_NODES, HD)
    eftf = eft.reshape(E_EDGES * HD)
    a_flat, parts = _phase1(dst, node2, eftf, target_idx)
    out2 = _phase2(parts[0], parts[1], target_idx)
    return (out2.reshape(B_TGT, H, D), a_flat.reshape(E_EDGES, H, 1))
